# Initial kernel scaffold; baseline (speedup 1.0000x reference)
import jax
import jax.numpy as jnp
from jax.experimental import pallas as pl


def kernel(x, root_nodes, root_biases, tree_indices, nodes, biases, leaf_nodes):
    raise NotImplementedError("write your pallas kernel here")


# speedup vs baseline: 794.3699x; 794.3699x over previous
import functools

import jax
import jax.numpy as jnp
from jax import lax
from jax.experimental import pallas as pl
from jax.experimental.pallas import tpu as pltpu
from jax.experimental.pallas import tpu_sc as plsc

NUM_TREES = 512
N_FEAT = 256
BATCH = 4096
N_LEVELS = 7
NLEAF = 256

NW = 32
RB = BATCH // NW
TPC = 16
NTC = NUM_TREES // TPC

LVL = [2 ** (j + 1) for j in range(N_LEVELS)]
OFF = [16 * sum(LVL[:j]) for j in range(N_LEVELS)]
NNODE = 16 * sum(LVL)
NI = NNODE + 16
NF = NNODE + 16 * NLEAF + 16


def _make_traversal():
    mesh = plsc.VectorSubcoreMesh(core_axis_name="c", subcore_axis_name="s")
    scratch = [
        pltpu.VMEM((RB * N_FEAT,), jnp.float32),
        pltpu.VMEM((NI,), jnp.int32),
        pltpu.VMEM((NF,), jnp.float32),
        pltpu.VMEM((RB, NUM_TREES), jnp.float32),
    ]

    @functools.partial(
        pl.kernel,
        out_type=jax.ShapeDtypeStruct((BATCH, NUM_TREES), jnp.float32),
        mesh=mesh,
        scratch_types=scratch,
        compiler_params=pltpu.CompilerParams(needs_layout_passes=False),
    )
    def k(x_hbm, ints_hbm, flts_hbm, out_hbm, xc, ints_loc, flts_loc, outc):
        w = lax.axis_index("s") * 2 + lax.axis_index("c")
        pltpu.sync_copy(x_hbm.at[pl.ds(w * (RB * N_FEAT), RB * N_FEAT)], xc)

        t_iota = lax.iota(jnp.int32, TPC)
        lvl_base = [OFF[j] + t_iota * LVL[j] for j in range(N_LEVELS)]
        leaf_base = NNODE + t_iota * NLEAF

        def chunk_body(tc, carry):
            pltpu.sync_copy(ints_hbm.at[pl.ds(tc * NI, NI)], ints_loc)
            pltpu.sync_copy(flts_hbm.at[pl.ds(tc * NF, NF)], flts_loc)
            rootf_v = ints_loc[pl.ds(NNODE, 16)]
            rootb_v = flts_loc[pl.ds(NNODE + 16 * NLEAF, 16)]
            out_col = tc * TPC + t_iota

            def row_body(r, rcarry):
                xb = r * N_FEAT
                xv = plsc.load_gather(xc, [rootf_v + xb])
                cur = (xv <= rootb_v).astype(jnp.int32)
                for j in range(N_LEVELS):
                    nidx = lvl_base[j] + cur
                    feat = plsc.load_gather(ints_loc, [nidx])
                    bias = plsc.load_gather(flts_loc, [nidx])
                    xvj = plsc.load_gather(xc, [feat + xb])
                    cur = cur + cur + (xvj <= bias).astype(jnp.int32)
                leafv = plsc.load_gather(flts_loc, [leaf_base + cur])
                plsc.store_scatter(
                    outc, [jnp.full((TPC,), r, jnp.int32), out_col], leafv)
                return rcarry

            lax.fori_loop(0, RB, row_body, 0)
            return carry

        lax.fori_loop(0, NTC, chunk_body, 0)
        pltpu.sync_copy(outc, out_hbm.at[pl.ds(w * RB, RB)])

    return k


_traverse = _make_traversal()


def kernel(x, root_nodes, root_biases, tree_indices, nodes, biases, leaf_nodes):
    del tree_indices
    ints = jnp.concatenate(
        [n.astype(jnp.int32).reshape(NTC, -1) for n in nodes]
        + [root_nodes.astype(jnp.int32).reshape(NTC, TPC)], axis=1)
    flts = jnp.concatenate(
        [b.reshape(NTC, -1) for b in biases]
        + [leaf_nodes.reshape(NTC, TPC * NLEAF),
           root_biases.reshape(NTC, TPC)], axis=1)
    out = _traverse(x.reshape(-1), ints.reshape(-1), flts.reshape(-1))
    return out.reshape(BATCH, NUM_TREES, 1)


# speedup vs baseline: 1750.9992x; 2.2043x over previous
import functools

import jax
import jax.numpy as jnp
from jax import lax
from jax.experimental import pallas as pl
from jax.experimental.pallas import tpu as pltpu
from jax.experimental.pallas import tpu_sc as plsc

NUM_TREES = 512
N_FEAT = 256
BATCH = 4096
N_LEVELS = 7
NLEAF = 256

K = 4
NW = 32
RB = BATCH // NW
TPC = 16
NTC = NUM_TREES // TPC

LVL = [2 ** (j + 1) for j in range(N_LEVELS)]
OFF = [16 * sum(LVL[:j]) for j in range(N_LEVELS)]
NNODE = 16 * sum(LVL)
NI = NNODE + 16
NF = NNODE + 16 * NLEAF + 16


def _make_traversal():
    mesh = plsc.VectorSubcoreMesh(core_axis_name="c", subcore_axis_name="s")
    scratch = [
        pltpu.VMEM((RB * N_FEAT,), jnp.float32),
        pltpu.VMEM((NI,), jnp.int32),
        pltpu.VMEM((NF,), jnp.float32),
        pltpu.VMEM((RB, NUM_TREES), jnp.float32),
    ]

    @functools.partial(
        pl.kernel,
        out_type=jax.ShapeDtypeStruct((BATCH, NUM_TREES), jnp.float32),
        mesh=mesh,
        scratch_types=scratch,
        compiler_params=pltpu.CompilerParams(needs_layout_passes=False),
    )
    def k(x_hbm, ints_hbm, flts_hbm, out_hbm, xc, ints_loc, flts_loc, outc):
        w = lax.axis_index("s") * 2 + lax.axis_index("c")
        pltpu.sync_copy(x_hbm.at[pl.ds(w * (RB * N_FEAT), RB * N_FEAT)], xc)

        t_iota = lax.iota(jnp.int32, TPC)
        lvl_base = [OFF[j] + t_iota * LVL[j] for j in range(N_LEVELS)]
        leaf_base = NNODE + t_iota * NLEAF

        def chunk_body(tc, carry):
            pltpu.sync_copy(ints_hbm.at[pl.ds(tc * NI, NI)], ints_loc)
            pltpu.sync_copy(flts_hbm.at[pl.ds(tc * NF, NF)], flts_loc)
            rootf_v = ints_loc[pl.ds(NNODE, 16)]
            rootb_v = flts_loc[pl.ds(NNODE + 16 * NLEAF, 16)]
            out_col = tc * TPC + t_iota

            def row_body(i, rcarry):
                xbs = [(i * K + kk) * N_FEAT for kk in range(K)]
                xvs = [plsc.load_gather(xc, [rootf_v + xb]) for xb in xbs]
                curs = [(xv <= rootb_v).astype(jnp.int32) for xv in xvs]
                for j in range(N_LEVELS):
                    nidxs = [lvl_base[j] + cur for cur in curs]
                    feats = [plsc.load_gather(ints_loc, [ni]) for ni in nidxs]
                    biass = [plsc.load_gather(flts_loc, [ni]) for ni in nidxs]
                    xvjs = [plsc.load_gather(xc, [f + xb])
                            for f, xb in zip(feats, xbs)]
                    curs = [c + c + (xvj <= b).astype(jnp.int32)
                            for c, xvj, b in zip(curs, xvjs, biass)]
                for kk in range(K):
                    leafv = plsc.load_gather(flts_loc, [leaf_base + curs[kk]])
                    plsc.store_scatter(
                        outc,
                        [jnp.full((TPC,), i * K + kk, jnp.int32), out_col],
                        leafv)
                return rcarry

            lax.fori_loop(0, RB // K, row_body, 0)
            return carry

        lax.fori_loop(0, NTC, chunk_body, 0)
        pltpu.sync_copy(outc, out_hbm.at[pl.ds(w * RB, RB)])

    return k


_traverse = _make_traversal()


def kernel(x, root_nodes, root_biases, tree_indices, nodes, biases, leaf_nodes):
    del tree_indices
    ints = jnp.concatenate(
        [n.astype(jnp.int32).reshape(NTC, -1) for n in nodes]
        + [root_nodes.astype(jnp.int32).reshape(NTC, TPC)], axis=1)
    flts = jnp.concatenate(
        [b.reshape(NTC, -1) for b in biases]
        + [leaf_nodes.reshape(NTC, TPC * NLEAF),
           root_biases.reshape(NTC, TPC)], axis=1)
    out = _traverse(x.reshape(-1), ints.reshape(-1), flts.reshape(-1))
    return out.reshape(BATCH, NUM_TREES, 1)


# speedup vs baseline: 2175.4030x; 1.2424x over previous
import functools

import jax
import jax.numpy as jnp
from jax import lax
from jax.experimental import pallas as pl
from jax.experimental.pallas import tpu as pltpu
from jax.experimental.pallas import tpu_sc as plsc

NUM_TREES = 512
N_FEAT = 256
BATCH = 4096
N_LEVELS = 7
NLEAF = 256

K = 8
NW = 32
RB = BATCH // NW
TPC = 16
NTC = NUM_TREES // TPC

LVL = [2 ** (j + 1) for j in range(N_LEVELS)]
OFF = [16 * sum(LVL[:j]) for j in range(N_LEVELS)]
NNODE = 16 * sum(LVL)
NI = NNODE + 16
NF = NNODE + 16 * NLEAF + 16


def _make_traversal():
    mesh = plsc.VectorSubcoreMesh(core_axis_name="c", subcore_axis_name="s")
    scratch = [
        pltpu.VMEM((RB * N_FEAT,), jnp.float32),
        pltpu.VMEM((NI,), jnp.int32),
        pltpu.VMEM((NF,), jnp.float32),
        pltpu.VMEM((RB, NUM_TREES), jnp.float32),
    ]

    @functools.partial(
        pl.kernel,
        out_type=jax.ShapeDtypeStruct((BATCH, NUM_TREES), jnp.float32),
        mesh=mesh,
        scratch_types=scratch,
        compiler_params=pltpu.CompilerParams(needs_layout_passes=False),
    )
    def k(x_hbm, ints_hbm, flts_hbm, out_hbm, xc, ints_loc, flts_loc, outc):
        w = lax.axis_index("s") * 2 + lax.axis_index("c")
        pltpu.sync_copy(x_hbm.at[pl.ds(w * (RB * N_FEAT), RB * N_FEAT)], xc)

        t_iota = lax.iota(jnp.int32, TPC)
        lvl_base = [OFF[j] + t_iota * LVL[j] for j in range(N_LEVELS)]
        leaf_base = NNODE + t_iota * NLEAF

        def chunk_body(tc, carry):
            pltpu.sync_copy(ints_hbm.at[pl.ds(tc * NI, NI)], ints_loc)
            pltpu.sync_copy(flts_hbm.at[pl.ds(tc * NF, NF)], flts_loc)
            rootf_v = ints_loc[pl.ds(NNODE, 16)]
            rootb_v = flts_loc[pl.ds(NNODE + 16 * NLEAF, 16)]
            out_col = tc * TPC + t_iota

            def row_body(i, rcarry):
                xbs = [(i * K + kk) * N_FEAT for kk in range(K)]
                xvs = [plsc.load_gather(xc, [rootf_v + xb]) for xb in xbs]
                curs = [(xv <= rootb_v).astype(jnp.int32) for xv in xvs]
                for j in range(N_LEVELS):
                    nidxs = [lvl_base[j] + cur for cur in curs]
                    feats = [plsc.load_gather(ints_loc, [ni]) for ni in nidxs]
                    biass = [plsc.load_gather(flts_loc, [ni]) for ni in nidxs]
                    xvjs = [plsc.load_gather(xc, [f + xb])
                            for f, xb in zip(feats, xbs)]
                    curs = [c + c + (xvj <= b).astype(jnp.int32)
                            for c, xvj, b in zip(curs, xvjs, biass)]
                for kk in range(K):
                    leafv = plsc.load_gather(flts_loc, [leaf_base + curs[kk]])
                    plsc.store_scatter(
                        outc,
                        [jnp.full((TPC,), i * K + kk, jnp.int32), out_col],
                        leafv)
                return rcarry

            lax.fori_loop(0, RB // K, row_body, 0)
            return carry

        lax.fori_loop(0, NTC, chunk_body, 0)
        pltpu.sync_copy(outc, out_hbm.at[pl.ds(w * RB, RB)])

    return k


_traverse = _make_traversal()


def kernel(x, root_nodes, root_biases, tree_indices, nodes, biases, leaf_nodes):
    del tree_indices
    ints = jnp.concatenate(
        [n.astype(jnp.int32).reshape(NTC, -1) for n in nodes]
        + [root_nodes.astype(jnp.int32).reshape(NTC, TPC)], axis=1)
    flts = jnp.concatenate(
        [b.reshape(NTC, -1) for b in biases]
        + [leaf_nodes.reshape(NTC, TPC * NLEAF),
           root_biases.reshape(NTC, TPC)], axis=1)
    out = _traverse(x.reshape(-1), ints.reshape(-1), flts.reshape(-1))
    return out.reshape(BATCH, NUM_TREES, 1)


# speedup vs baseline: 3330.9320x; 1.5312x over previous
import functools

import jax
import jax.numpy as jnp
from jax import lax
from jax.experimental import pallas as pl
from jax.experimental.pallas import tpu as pltpu
from jax.experimental.pallas import tpu_sc as plsc

NUM_TREES = 512
N_FEAT = 256
BATCH = 4096
N_LEVELS = 7
NLEAF = 256

K = 8
NW = 32
RB = BATCH // NW
TPC = 16
NTC = NUM_TREES // TPC

LVL = [2 ** (j + 1) for j in range(N_LEVELS)]


def _make_traversal():
    mesh = plsc.VectorSubcoreMesh(core_axis_name="c", subcore_axis_name="s")
    buf_set = (
        [pltpu.VMEM((TPC * s,), jnp.int32) for s in LVL]
        + [pltpu.VMEM((TPC * s,), jnp.float32) for s in LVL]
        + [pltpu.VMEM((TPC * NLEAF,), jnp.float32)]
    )
    scratch = (
        [
            pltpu.VMEM((RB * N_FEAT,), jnp.float32),
            pltpu.VMEM((NUM_TREES,), jnp.int32),
            pltpu.VMEM((NUM_TREES,), jnp.float32),
            pltpu.VMEM((RB, NUM_TREES), jnp.float32),
            pltpu.SemaphoreType.DMA,
            pltpu.SemaphoreType.DMA,
            pltpu.SemaphoreType.DMA,
        ]
        + buf_set + buf_set
    )

    @functools.partial(
        pl.kernel,
        out_type=jax.ShapeDtypeStruct((BATCH, NUM_TREES), jnp.float32),
        mesh=mesh,
        scratch_types=scratch,
        compiler_params=pltpu.CompilerParams(needs_layout_passes=False),
    )
    def k(x_hbm, rn_hbm, rb_hbm,
          n0, n1, n2, n3, n4, n5, n6,
          c0, c1, c2, c3, c4, c5, c6,
          leaf_hbm, out_hbm,
          xc, rootn_loc, rootb_loc, outc, semx, sema, semb, *bufs):
        w = lax.axis_index("s") * 2 + lax.axis_index("c")
        n_hbm = [n0, n1, n2, n3, n4, n5, n6]
        b_hbm = [c0, c1, c2, c3, c4, c5, c6]
        set_a, set_b = bufs[:15], bufs[15:]

        def start_set(tc, S, sem):
            for j in range(N_LEVELS):
                sz = TPC * LVL[j]
                pltpu.async_copy(n_hbm[j].at[pl.ds(tc * sz, sz)], S[j], sem)
                pltpu.async_copy(
                    b_hbm[j].at[pl.ds(tc * sz, sz)], S[N_LEVELS + j], sem)
            pltpu.async_copy(
                leaf_hbm.at[pl.ds(tc * (TPC * NLEAF), TPC * NLEAF)],
                S[2 * N_LEVELS], sem)

        def wait_set(S, sem):
            for j in range(N_LEVELS):
                sz = TPC * LVL[j]
                pltpu.make_async_copy(
                    n_hbm[j].at[pl.ds(0, sz)], S[j], sem).wait()
                pltpu.make_async_copy(
                    b_hbm[j].at[pl.ds(0, sz)], S[N_LEVELS + j], sem).wait()
            pltpu.make_async_copy(
                leaf_hbm.at[pl.ds(0, TPC * NLEAF)], S[2 * N_LEVELS], sem
            ).wait()

        xcopy = pltpu.async_copy(
            x_hbm.at[pl.ds(w * (RB * N_FEAT), RB * N_FEAT)], xc, semx)
        rncopy = pltpu.async_copy(rn_hbm, rootn_loc, semx)
        rbcopy = pltpu.async_copy(rb_hbm, rootb_loc, semx)
        start_set(0, set_a, sema)
        xcopy.wait()
        rncopy.wait()
        rbcopy.wait()

        t_iota = lax.iota(jnp.int32, TPC)
        lvl_base = [t_iota * LVL[j] for j in range(N_LEVELS)]
        leaf_base = t_iota * NLEAF

        def process(tc, S):
            rootf_v = rootn_loc[pl.ds(tc * TPC, TPC)]
            rootb_v = rootb_loc[pl.ds(tc * TPC, TPC)]
            out_col = tc * TPC + t_iota

            def row_body(i, rcarry):
                xbs = [(i * K + kk) * N_FEAT for kk in range(K)]
                xvs = [plsc.load_gather(xc, [rootf_v + xb]) for xb in xbs]
                curs = [(xv <= rootb_v).astype(jnp.int32) for xv in xvs]
                for j in range(N_LEVELS):
                    nidxs = [lvl_base[j] + cur for cur in curs]
                    feats = [plsc.load_gather(S[j], [ni]) for ni in nidxs]
                    biass = [plsc.load_gather(S[N_LEVELS + j], [ni])
                             for ni in nidxs]
                    xvjs = [plsc.load_gather(xc, [f + xb])
                            for f, xb in zip(feats, xbs)]
                    curs = [c + c + (xvj <= b).astype(jnp.int32)
                            for c, xvj, b in zip(curs, xvjs, biass)]
                for kk in range(K):
                    leafv = plsc.load_gather(
                        S[2 * N_LEVELS], [leaf_base + curs[kk]])
                    plsc.store_scatter(
                        outc,
                        [jnp.full((TPC,), i * K + kk, jnp.int32), out_col],
                        leafv)
                return rcarry

            lax.fori_loop(0, RB // K, row_body, 0)

        def super_body(i, carry):
            wait_set(set_a, sema)
            start_set(2 * i + 1, set_b, semb)
            process(2 * i, set_a)
            wait_set(set_b, semb)

            @pl.when(i < NTC // 2 - 1)
            def _prefetch_next():
                start_set(2 * i + 2, set_a, sema)

            process(2 * i + 1, set_b)
            return carry

        lax.fori_loop(0, NTC // 2, super_body, 0)
        pltpu.sync_copy(outc, out_hbm.at[pl.ds(w * RB, RB)])

    return k


_traverse = _make_traversal()


def kernel(x, root_nodes, root_biases, tree_indices, nodes, biases, leaf_nodes):
    del tree_indices
    out = _traverse(
        x.reshape(-1),
        root_nodes.astype(jnp.int32),
        root_biases,
        *[n.astype(jnp.int32) for n in nodes],
        *biases,
        leaf_nodes.reshape(-1),
    )
    return out.reshape(BATCH, NUM_TREES, 1)
